# Initial kernel scaffold; baseline (speedup 1.0000x reference)
import jax
import jax.numpy as jnp
from jax.experimental import pallas as pl


def kernel(node_features, edge_node_indices, edge_features, W1, b1, W2, b2):
    raise NotImplementedError("write your pallas kernel here")


# speedup vs baseline: 4.8569x; 4.8569x over previous
import functools

import jax
import jax.numpy as jnp
from jax import lax
from jax.experimental import pallas as pl
from jax.experimental.pallas import tpu as pltpu
from jax.experimental.pallas import tpu_sc as plsc

N_NODES = 10000
N_EDGES = 320000
D_NODE = 128
D_EDGE = 16
MSG = 32

NC = 2
NS = 16
NW = NC * NS

SUB = 125
EPW = N_EDGES // NW
ROWS_PW = EPW // SUB
R_CHUNK = 8
C_EDGES = R_CHUNK * SUB
N_CHUNK = ROWS_PW // R_CHUNK
NPZ = 624


def _proj_nodes_body(x_ref, w0_ref, w1_ref, p0_ref, p1_ref):
    x = x_ref[...]
    p0_ref[...] = jnp.dot(x, w0_ref[...], preferred_element_type=jnp.float32)
    p1_ref[...] = jnp.dot(x, w1_ref[...], preferred_element_type=jnp.float32)


def _proj_nodes(x, w0T, w1T):
    blk = 1000
    grid = N_NODES // blk
    return pl.pallas_call(
        _proj_nodes_body,
        grid=(grid,),
        in_specs=[
            pl.BlockSpec((blk, D_NODE), lambda i: (i, 0)),
            pl.BlockSpec((D_NODE, MSG), lambda i: (0, 0)),
            pl.BlockSpec((D_NODE, MSG), lambda i: (0, 0)),
        ],
        out_specs=[
            pl.BlockSpec((blk, MSG), lambda i: (i, 0)),
            pl.BlockSpec((blk, MSG), lambda i: (i, 0)),
        ],
        out_shape=[
            jax.ShapeDtypeStruct((N_NODES, MSG), jnp.float32),
            jax.ShapeDtypeStruct((N_NODES, MSG), jnp.float32),
        ],
    )(x, w0T, w1T)


def _proj_edges_body(ef_ref, wc_ref, b1_ref, ep_ref):
    ep_ref[...] = (
        jnp.dot(ef_ref[...], wc_ref[...], preferred_element_type=jnp.float32)
        + b1_ref[...]
    )


def _proj_edges(ef, wcT, b1r):
    blk = 2000
    grid = N_EDGES // blk
    return pl.pallas_call(
        _proj_edges_body,
        grid=(grid,),
        in_specs=[
            pl.BlockSpec((blk, D_EDGE), lambda i: (i, 0)),
            pl.BlockSpec((D_EDGE, MSG), lambda i: (0, 0)),
            pl.BlockSpec((1, MSG), lambda i: (0, 0)),
        ],
        out_specs=pl.BlockSpec((blk, MSG), lambda i: (i, 0)),
        out_shape=jax.ShapeDtypeStruct((N_EDGES, MSG), jnp.float32),
    )(ef, wcT, b1r)


def _sc_body(p0_hbm, p1_hbm, ep_hbm, i0_hbm, i1_hbm, out_hbm,
             i0_v, i1_v, msg_v, g0_v, g1_v, acc_sh, sem):
    cid = lax.axis_index("c")
    sid = lax.axis_index("s")
    wid = sid * NC + cid

    def zrow(r, carry):
        g0_v[r, pl.ds(0, 16)] = jnp.zeros((16,), jnp.float32)
        g0_v[r, pl.ds(16, 16)] = jnp.zeros((16,), jnp.float32)
        return carry

    lax.fori_loop(0, NPZ + 16, zrow, 0)
    pltpu.sync_copy(g0_v.at[pl.ds(0, NPZ)], acc_sh.at[pl.ds(sid * NPZ, NPZ)])

    @pl.when(sid == NS - 1)
    def _zero_tail():
        pltpu.sync_copy(g0_v.at[pl.ds(0, 16)], acc_sh.at[pl.ds(NS * NPZ, 16)])

    plsc.subcore_barrier()

    def chunk(ci, carry):
        rbase = wid * ROWS_PW + ci * R_CHUNK
        ebase = wid * EPW + ci * C_EDGES
        pltpu.sync_copy(i0_hbm.at[pl.ds(rbase, R_CHUNK)], i0_v)
        pltpu.sync_copy(i1_hbm.at[pl.ds(rbase, R_CHUNK)], i1_v)
        cps = [pltpu.async_copy(ep_hbm.at[pl.ds(ebase, C_EDGES)], msg_v, sem)]
        for j in range(R_CHUNK):
            dst = pl.ds(j * SUB, SUB)
            cps.append(pltpu.async_copy(p0_hbm.at[i0_v.at[j]], g0_v.at[dst], sem))
            cps.append(pltpu.async_copy(p1_hbm.at[i1_v.at[j]], g1_v.at[dst], sem))
        for c in cps:
            c.wait()

        def rowf(r, rcarry):
            for off in (0, 16):
                s = pl.ds(off, 16)
                msg_v[r, s] = jnp.maximum(
                    msg_v[r, s] + g0_v[r, s] + g1_v[r, s], 0.0
                )
            return rcarry

        lax.fori_loop(0, C_EDGES, rowf, 0)
        for j in range(R_CHUNK):
            pltpu.sync_copy(
                msg_v.at[pl.ds(j * SUB, SUB)], acc_sh.at[i0_v.at[j]], add=True
            )
        return carry

    lax.fori_loop(0, N_CHUNK, chunk, 0)
    plsc.subcore_barrier()
    pltpu.sync_copy(
        acc_sh.at[pl.ds(sid * NPZ, NPZ)], out_hbm.at[cid, pl.ds(sid * NPZ, NPZ)]
    )

    @pl.when(sid == NS - 1)
    def _write_tail():
        pltpu.sync_copy(
            acc_sh.at[pl.ds(NS * NPZ, 16)], out_hbm.at[cid, pl.ds(NS * NPZ, 16)]
        )


def _sc_gather_scatter(P0, P1, EP, i0, i1):
    mesh = plsc.VectorSubcoreMesh(core_axis_name="c", subcore_axis_name="s")
    return pl.kernel(
        _sc_body,
        out_type=jax.ShapeDtypeStruct((NC, N_NODES, MSG), jnp.float32),
        mesh=mesh,
        compiler_params=pltpu.CompilerParams(use_tc_tiling_on_sc=False),
        scratch_types=[
            pltpu.VMEM((R_CHUNK, SUB), jnp.int32),
            pltpu.VMEM((R_CHUNK, SUB), jnp.int32),
            pltpu.VMEM((C_EDGES, MSG), jnp.float32),
            pltpu.VMEM((C_EDGES, MSG), jnp.float32),
            pltpu.VMEM((C_EDGES, MSG), jnp.float32),
            pltpu.VMEM_SHARED((N_NODES, MSG), jnp.float32),
            pltpu.SemaphoreType.DMA,
        ],
    )(P0, P1, EP, i0, i1)


def _final_body(x_ref, part_ref, w2a_ref, w2b_ref, b2_ref, out_ref):
    acc = part_ref[0] + part_ref[1]
    o = (
        jnp.dot(x_ref[...], w2a_ref[...], preferred_element_type=jnp.float32)
        + jnp.dot(acc, w2b_ref[...], preferred_element_type=jnp.float32)
        + b2_ref[...]
    )
    out_ref[...] = jnp.maximum(o, 0.0)


def _final(x, part, w2aT, w2bT, b2r):
    blk = 1000
    grid = N_NODES // blk
    return pl.pallas_call(
        _final_body,
        grid=(grid,),
        in_specs=[
            pl.BlockSpec((blk, D_NODE), lambda i: (i, 0)),
            pl.BlockSpec((NC, blk, MSG), lambda i: (0, i, 0)),
            pl.BlockSpec((D_NODE, D_NODE), lambda i: (0, 0)),
            pl.BlockSpec((MSG, D_NODE), lambda i: (0, 0)),
            pl.BlockSpec((1, D_NODE), lambda i: (0, 0)),
        ],
        out_specs=pl.BlockSpec((blk, D_NODE), lambda i: (i, 0)),
        out_shape=jax.ShapeDtypeStruct((N_NODES, D_NODE), jnp.float32),
    )(x, part, w2aT, w2bT, b2r)


def kernel(node_features, edge_node_indices, edge_features, W1, b1, W2, b2):
    x = node_features
    n0 = edge_node_indices[0].astype(jnp.int32)
    n1 = edge_node_indices[1].astype(jnp.int32)
    w0T = W1[:, :D_NODE].T
    w1T = W1[:, D_NODE:2 * D_NODE].T
    wcT = W1[:, 2 * D_NODE:].T
    w2aT = W2[:, :D_NODE].T
    w2bT = W2[:, D_NODE:].T
    b1r = b1.reshape(1, MSG)
    b2r = b2.reshape(1, D_NODE)

    P0, P1 = _proj_nodes(x, w0T, w1T)
    EP = _proj_edges(edge_features, wcT, b1r)
    i0 = n0.reshape(N_EDGES // SUB, SUB)
    i1 = n1.reshape(N_EDGES // SUB, SUB)
    part = _sc_gather_scatter(P0, P1, EP, i0, i1)
    return _final(x, part, w2aT, w2bT, b2r)
